# baseline (speedup 1.0000x reference)
import functools

import jax
import jax.numpy as jnp
from jax import lax
from jax.experimental import pallas as pl
from jax.experimental.pallas import tpu as pltpu
from jax.experimental.pallas import tpu_sc as plsc

_N_CLS = 1000
_D = 512
_P = 5
_C = 5
_T = 67
_M = _P + _C + _T

_PW = _P * _D
_CW = _C * _D
_TW = _T * _D
_MW = _M * _D

_info = plsc.get_sparse_core_info()
_NC = _info.num_cores
_NS = _info.num_subcores
_NW = _NC * _NS

_BASE = _N_CLS // _NW
_EXTRA = _N_CLS % _NW

_DEPTH = 8

_mesh = plsc.VectorSubcoreMesh(core_axis_name="c", subcore_axis_name="s")


@functools.partial(
    pl.kernel,
    mesh=_mesh,
    out_type=jax.ShapeDtypeStruct((_N_CLS * _MW,), jnp.float32),
    scratch_types=[
        pltpu.SemaphoreType.DMA,
    ],
)
def _prompt_concat(prefix_hbm, ctx_hbm, token_hbm, out_hbm, sem):
    core = lax.axis_index("c")
    sub = lax.axis_index("s")
    wid = sub * _NC + core
    cnt = _BASE + jnp.where(wid < _EXTRA, 1, 0)
    start = _BASE * wid + jnp.minimum(wid, _EXTRA)

    def fire(c):
        o = c * _MW
        pltpu.async_copy(prefix_hbm, out_hbm.at[pl.ds(o, _PW)], sem)
        pltpu.async_copy(
            ctx_hbm.at[pl.ds(c * _CW, _CW)],
            out_hbm.at[pl.ds(o + _PW, _CW)],
            sem,
        )
        pltpu.async_copy(
            token_hbm.at[pl.ds(c * _TW, _TW)],
            out_hbm.at[pl.ds(o + _PW + _CW, _TW)],
            sem,
        )

    def drain_one():
        pltpu.make_async_copy(prefix_hbm, out_hbm.at[pl.ds(0, _PW)], sem).wait()
        pltpu.make_async_copy(
            ctx_hbm.at[pl.ds(0, _CW)], out_hbm.at[pl.ds(0, _CW)], sem
        ).wait()
        pltpu.make_async_copy(
            token_hbm.at[pl.ds(0, _TW)], out_hbm.at[pl.ds(0, _TW)], sem
        ).wait()

    def body(i, carry):
        fire(start + i)

        @pl.when(i >= _DEPTH)
        def _():
            drain_one()

        return carry

    lax.fori_loop(0, cnt, body, 0)

    def tail(i, carry):
        drain_one()
        return carry

    lax.fori_loop(0, _DEPTH, tail, 0)


def kernel(prefix, ctx, token):
    out = _prompt_concat(
        prefix.reshape(_PW),
        ctx.reshape(_N_CLS * _CW),
        token.reshape(_N_CLS * _TW),
    )
    return out.reshape(_N_CLS, _M, _D)


# speedup vs baseline: 4.4182x; 4.4182x over previous
import functools

import jax
import jax.numpy as jnp
from jax import lax
from jax.experimental import pallas as pl
from jax.experimental.pallas import tpu as pltpu
from jax.experimental.pallas import tpu_sc as plsc

_N_CLS = 1000
_D = 512
_P = 5
_C = 5
_T = 67
_M = _P + _C + _T

_PW = _P * _D
_CW = _C * _D
_TW = _T * _D
_MW = _M * _D

_info = plsc.get_sparse_core_info()
_NC = _info.num_cores
_NS = _info.num_subcores
_NW = _NC * _NS

_BASE = _N_CLS // _NW
_EXTRA = _N_CLS % _NW

_mesh = plsc.VectorSubcoreMesh(core_axis_name="c", subcore_axis_name="s")


@functools.partial(
    pl.kernel,
    mesh=_mesh,
    out_type=jax.ShapeDtypeStruct((_N_CLS * _MW,), jnp.float32),
    scratch_types=[
        pltpu.VMEM((2 * _MW,), jnp.float32),
        pltpu.SemaphoreType.DMA,
        pltpu.SemaphoreType.DMA,
    ],
)
def _prompt_concat(prefix_hbm, ctx_hbm, token_hbm, out_hbm, buf, in_sem, out_sem):
    core = lax.axis_index("c")
    sub = lax.axis_index("s")
    wid = sub * _NC + core
    cnt = _BASE + jnp.where(wid < _EXTRA, 1, 0)
    start = _BASE * wid + jnp.minimum(wid, _EXTRA)

    pltpu.sync_copy(prefix_hbm, buf.at[pl.ds(0, _PW)])
    pltpu.sync_copy(prefix_hbm, buf.at[pl.ds(_MW, _PW)])

    def slot_off(i):
        return pl.multiple_of(lax.rem(i, 2) * _MW, _MW)

    def fire_load(i):
        c = start + i
        o = slot_off(i)
        pltpu.async_copy(
            ctx_hbm.at[pl.ds(c * _CW, _CW)], buf.at[pl.ds(o + _PW, _CW)], in_sem
        )
        pltpu.async_copy(
            token_hbm.at[pl.ds(c * _TW, _TW)],
            buf.at[pl.ds(o + _PW + _CW, _TW)],
            in_sem,
        )

    def wait_load():
        pltpu.make_async_copy(
            ctx_hbm.at[pl.ds(0, _CW)], buf.at[pl.ds(_PW, _CW)], in_sem
        ).wait()
        pltpu.make_async_copy(
            token_hbm.at[pl.ds(0, _TW)], buf.at[pl.ds(_PW + _CW, _TW)], in_sem
        ).wait()

    def fire_store(i):
        c = start + i
        o = slot_off(i)
        pltpu.async_copy(
            buf.at[pl.ds(o, _MW)], out_hbm.at[pl.ds(c * _MW, _MW)], out_sem
        )

    def wait_store():
        pltpu.make_async_copy(
            buf.at[pl.ds(0, _MW)], out_hbm.at[pl.ds(0, _MW)], out_sem
        ).wait()

    fire_load(0)

    def body(i, carry):
        @pl.when(i + 1 < cnt)
        def _():
            @pl.when(i >= 1)
            def _():
                wait_store()

            fire_load(i + 1)

        wait_load()
        fire_store(i)
        return carry

    lax.fori_loop(0, cnt, body, 0)

    wait_store()
    wait_store()


def kernel(prefix, ctx, token):
    out = _prompt_concat(
        prefix.reshape(_PW),
        ctx.reshape(_N_CLS * _CW),
        token.reshape(_N_CLS * _TW),
    )
    return out.reshape(_N_CLS, _M, _D)


# speedup vs baseline: 10.1562x; 2.2987x over previous
import functools

import jax
import jax.numpy as jnp
from jax import lax
from jax.experimental import pallas as pl
from jax.experimental.pallas import tpu as pltpu
from jax.experimental.pallas import tpu_sc as plsc

_N_CLS = 1000
_D = 512
_P = 5
_C = 5
_T = 67
_M = _P + _C + _T

_info = plsc.get_sparse_core_info()
_NC = _info.num_cores
_NS = _info.num_subcores
_NW = _NC * _NS

_BASE = _N_CLS // _NW
_EXTRA = _N_CLS % _NW

_LANES = 16
_JROW = _D // _LANES

_mesh = plsc.VectorSubcoreMesh(core_axis_name="c", subcore_axis_name="s")


@functools.partial(
    pl.kernel,
    mesh=_mesh,
    out_type=jax.ShapeDtypeStruct((_N_CLS, _M, _D), jnp.float32),
    scratch_types=[
        pltpu.VMEM((_C, _D), jnp.float32),
        pltpu.VMEM((_T, _D), jnp.float32),
        pltpu.VMEM((2, _M, _D), jnp.float32),
        pltpu.SemaphoreType.DMA,
        pltpu.SemaphoreType.DMA,
    ],
    compiler_params=pltpu.CompilerParams(use_tc_tiling_on_sc=True),
)
def _prompt_concat(
    prefix_hbm, ctx_hbm, token_hbm, out_hbm, ctx_buf, tok_buf, out_buf, in_sem, out_sem
):
    core = lax.axis_index("c")
    sub = lax.axis_index("s")
    wid = sub * _NC + core
    cnt = _BASE + jnp.where(wid < _EXTRA, 1, 0)
    start = _BASE * wid + jnp.minimum(wid, _EXTRA)

    pltpu.sync_copy(prefix_hbm, out_buf.at[0, pl.ds(0, _P)])
    pltpu.sync_copy(prefix_hbm, out_buf.at[1, pl.ds(0, _P)])

    def fire_load(c):
        pltpu.async_copy(ctx_hbm.at[c], ctx_buf, in_sem)
        pltpu.async_copy(token_hbm.at[c], tok_buf, in_sem)

    def wait_load():
        pltpu.make_async_copy(ctx_hbm.at[0], ctx_buf, in_sem).wait()
        pltpu.make_async_copy(token_hbm.at[0], tok_buf, in_sem).wait()

    def wait_store():
        pltpu.make_async_copy(out_buf.at[0], out_hbm.at[0], out_sem).wait()

    def shuffle(slot):
        for r in range(_C):
            for j in range(_JROW):
                sl = pl.ds(j * _LANES, _LANES)
                out_buf[slot, _P + r, sl] = ctx_buf[r, sl]

        def trow(s, carry):
            for j in range(_JROW):
                sl = pl.ds(j * _LANES, _LANES)
                out_buf[slot, _P + _C + s, sl] = tok_buf[s, sl]
            return carry

        lax.fori_loop(0, _T, trow, 0)

    fire_load(start)

    def body(i, carry):
        slot = lax.rem(i, 2)
        wait_load()

        @pl.when(i >= 2)
        def _():
            wait_store()

        shuffle(slot)
        pltpu.async_copy(out_buf.at[slot], out_hbm.at[start + i], out_sem)

        @pl.when(i + 1 < cnt)
        def _():
            fire_load(start + i + 1)

        return carry

    lax.fori_loop(0, cnt, body, 0)

    wait_store()
    wait_store()


def kernel(prefix, ctx, token):
    return _prompt_concat(prefix, ctx, token)


# speedup vs baseline: 47.9980x; 4.7260x over previous
import functools

import jax
import jax.numpy as jnp
from jax import lax
from jax.experimental import pallas as pl
from jax.experimental.pallas import tpu as pltpu
from jax.experimental.pallas import tpu_sc as plsc

_N_CLS = 1000
_D = 512
_P = 5
_C = 5
_T = 67
_M = _P + _C + _T

_info = plsc.get_sparse_core_info()
_NC = _info.num_cores
_NS = _info.num_subcores
_NW = _NC * _NS

_ROWS = 32
_LAST = _N_CLS - _ROWS

_K = 3
_NCHUNK = _T // _K

_LANES = 16
_JROW = _D // _LANES

_mesh = plsc.VectorSubcoreMesh(core_axis_name="c", subcore_axis_name="s")


@functools.partial(
    pl.kernel,
    mesh=_mesh,
    out_type=jax.ShapeDtypeStruct((_M, _N_CLS, _D), jnp.float32),
    scratch_types=[
        pltpu.VMEM((2, _K, _ROWS, _D), jnp.float32),
        pltpu.VMEM((_P, _D), jnp.float32),
        pltpu.VMEM((_P, 8, _D), jnp.float32),
        pltpu.SemaphoreType.DMA,
        pltpu.SemaphoreType.DMA,
    ],
)
def _prompt_concat(
    prefix_hbm, ctx_hbm, token_hbm, out_hbm, buf, pbuf, prep, in_sem, out_sem
):
    core = lax.axis_index("c")
    sub = lax.axis_index("s")
    wid = sub * _NC + core
    start = pl.multiple_of(jnp.minimum(_ROWS * wid, _LAST), 8)
    rows = pl.ds(start, _ROWS)

    def wait_in(k):
        pltpu.make_async_copy(
            token_hbm.at[pl.ds(0, k), pl.ds(0, _ROWS)],
            buf.at[0, pl.ds(0, k)],
            in_sem,
        ).wait()

    def wait_out(k):
        pltpu.make_async_copy(
            buf.at[0, pl.ds(0, k)],
            out_hbm.at[pl.ds(0, k), pl.ds(0, _ROWS)],
            out_sem,
        ).wait()

    pltpu.async_copy(prefix_hbm, pbuf, in_sem)

    pltpu.async_copy(ctx_hbm.at[pl.ds(0, 3), rows], buf.at[0], in_sem)
    pltpu.async_copy(ctx_hbm.at[pl.ds(3, 2), rows], buf.at[1, pl.ds(0, 2)], in_sem)
    pltpu.make_async_copy(prefix_hbm, pbuf, in_sem).wait()
    wait_in(3)
    pltpu.async_copy(buf.at[0], out_hbm.at[pl.ds(_P, 3), rows], out_sem)
    wait_in(2)
    pltpu.async_copy(
        buf.at[1, pl.ds(0, 2)], out_hbm.at[pl.ds(_P + 3, 2), rows], out_sem
    )
    wait_out(3)
    wait_out(2)

    pltpu.async_copy(token_hbm.at[pl.ds(0, _K), rows], buf.at[0], in_sem)

    def body(i, carry):
        slot = lax.rem(i, 2)
        wait_in(_K)
        pltpu.async_copy(
            buf.at[slot],
            out_hbm.at[pl.ds(_P + _C + _K * i, _K), rows],
            out_sem,
        )

        @pl.when(i + 1 < _NCHUNK)
        def _():
            @pl.when(i >= 1)
            def _():
                wait_out(_K)

            pltpu.async_copy(
                token_hbm.at[pl.ds(_K * (i + 1), _K), rows],
                buf.at[1 - slot],
                in_sem,
            )

        return carry

    lax.fori_loop(0, _NCHUNK, body, 0)

    wait_out(_K)

    pltpu.async_copy(
        token_hbm.at[pl.ds(_K * _NCHUNK, 1), rows], buf.at[0, pl.ds(0, 1)], in_sem
    )
    wait_in(1)
    pltpu.async_copy(
        buf.at[0, pl.ds(0, 1)], out_hbm.at[pl.ds(_M - 1, 1), rows], out_sem
    )

    for r in range(_P):

        def fill(j, carry):
            sl = pl.ds(j * _LANES, _LANES)
            v = pbuf[r, sl]
            for r2 in range(8):
                prep[r, r2, sl] = v
            return carry

        lax.fori_loop(0, _JROW, fill, 0)
        for k in range(4):
            pltpu.async_copy(
                prep.at[r],
                out_hbm.at[r, pl.ds(start + 8 * k, 8)],
                out_sem,
            )

    wait_out(_K)
    wait_out(1)
    for r in range(_P):
        for k in range(4):
            pltpu.make_async_copy(
                prep.at[r], out_hbm.at[r, pl.ds(start, 8)], out_sem
            ).wait()


def kernel(prefix, ctx, token):
    out_t = _prompt_concat(
        prefix,
        ctx.transpose(1, 0, 2),
        token.transpose(1, 0, 2),
    )
    return out_t.transpose(1, 0, 2)
